# baseline (speedup 1.0000x reference)
import jax
import jax.numpy as jnp
from jax import lax
from jax.experimental import pallas as pl
from jax.experimental.pallas import tpu as pltpu
from jax.experimental.pallas import tpu_sc as plsc

_D = 64
_SEQ = 200
_BATCH = 4096
_NW = 32
_BB = _BATCH // _NW
_NBUF = 4
_NITER = _SEQ // _NBUF


def _pos_encoding(max_len, d_model):
    angle = jnp.arange(d_model, dtype=jnp.float32)
    angle = 10000.0 ** (2.0 * (angle / d_model))
    angle = jnp.arange(max_len, dtype=jnp.float32)[:, None] / angle
    values = jnp.stack([jnp.sin(angle[:, 0::2]), jnp.cos(angle[:, 1::2])], axis=2)
    return jnp.reshape(values, (values.shape[0], -1)).astype(jnp.float32)


def _sc_embed(seqT_hbm, pos_hbm, table_hbm, out_hbm, pos_v, idx_v,
              rows_g, sw, rows_w, isems, gsems, wsems):
    wid = lax.axis_index("s") * 2 + lax.axis_index("c")
    base_b = wid * _BB
    pltpu.sync_copy(pos_hbm, pos_v)

    iota16 = lax.iota(jnp.int32, 16)

    def fire_idx(s, b):
        pltpu.async_copy(seqT_hbm.at[s, pl.ds(base_b, _BB)], idx_v[b], isems[b])

    def prep(s, b):
        pltpu.make_async_copy(seqT_hbm.at[s, pl.ds(base_b, _BB)], idx_v[b],
                              isems[b]).wait()

        def enq(t, _):
            vec = idx_v[b][pl.ds(t * 16, 16)]
            for j in range(16):
                k = t * 16 + j
                pltpu.async_copy(table_hbm.at[vec[j]], rows_g[b].at[k],
                                 gsems[b])
            return 0

        lax.fori_loop(0, _BB // 16, enq, 0)

    def drain_rows(b):
        def drn(t, _):
            for j in range(8):
                pltpu.make_async_copy(table_hbm.at[0], rows_g[b].at[0],
                                      gsems[b]).wait()
            return 0

        lax.fori_loop(0, _BB // 8, drn, 0)

    def madd(s, b):
        pr = [pos_v[pl.ds(s * _D + i * 16, 16)] for i in range(4)]

        @plsc.parallel_loop(0, _BB)
        def _rows(r):
            r64 = r * 64
            for i in range(4):
                dvec = iota16 + (i * 16)
                val = rows_g[b][r, pl.ds(i * 16, 16)] * 8.0 + pr[i]
                addr = r64 + ((dvec + r) & 63)
                plsc.store_scatter(sw, [addr], val)

        @plsc.parallel_loop(0, _D)
        def _dims(d):
            for g in range(8):
                bvec = iota16 + (g * 16)
                addr = bvec * 64 + ((bvec + d) & 63)
                val = plsc.load_gather(sw, [addr])
                rows_w[b][d // 8, d % 8, pl.ds(g * 16, 16)] = val

    def fire_write(s, b):
        pltpu.async_copy(rows_w[b], out_hbm.at[pl.ds(s * 8, 8), wid], wsems[b])

    def wait_write(s, b):
        pltpu.make_async_copy(rows_w[b], out_hbm.at[pl.ds(s * 8, 8), wid],
                              wsems[b]).wait()

    for b in range(_NBUF):
        fire_idx(b, b)
    prep(0, 0)
    prep(1, 1)

    def ring_iter(g, _):
        for b in range(_NBUF):
            s = g * _NBUF + b

            @pl.when(s >= _NBUF)
            def _():
                wait_write(s - _NBUF, b)

            drain_rows(b)
            madd(s, b)
            fire_write(s, b)

            @pl.when(s + 2 < _SEQ)
            def _():
                prep(s + 2, (b + 2) % _NBUF)

            @pl.when(s + _NBUF < _SEQ)
            def _():
                fire_idx(s + _NBUF, b)
        return 0

    lax.fori_loop(0, _NITER, ring_iter, 0)
    for b in range(_NBUF):
        wait_write(_SEQ - _NBUF + b, b)


@jax.jit
def _embed(sequences, table):
    pos = _pos_encoding(_SEQ, _D).reshape(_SEQ * _D)
    seq_t = sequences.astype(jnp.int32).T
    mesh = plsc.VectorSubcoreMesh(core_axis_name="c", subcore_axis_name="s")
    out = pl.kernel(
        _sc_embed,
        out_type=jax.ShapeDtypeStruct((_SEQ * 8, _NW, 8, 128), jnp.float32),
        mesh=mesh,
        scratch_types=[
            pltpu.VMEM((_SEQ * _D,), jnp.float32),
            [pltpu.VMEM((_BB,), jnp.int32) for _ in range(_NBUF)],
            [pltpu.VMEM((_BB, _D), jnp.float32) for _ in range(_NBUF)],
            pltpu.VMEM((_BB * 64,), jnp.float32),
            [pltpu.VMEM((8, 8, 128), jnp.float32) for _ in range(_NBUF)],
            [pltpu.SemaphoreType.DMA for _ in range(_NBUF)],
            [pltpu.SemaphoreType.DMA for _ in range(_NBUF)],
            [pltpu.SemaphoreType.DMA for _ in range(_NBUF)],
        ],
        compiler_params=pltpu.CompilerParams(use_tc_tiling_on_sc=True,
                                             needs_layout_passes=False),
    )(seq_t, pos, table)
    out5 = out.reshape(_SEQ, 8, _NW, 8, 128)
    return out5.transpose(2, 4, 0, 1, 3).reshape(_BATCH, _SEQ, _D)


def kernel(sequences, table):
    return _embed(sequences, table)


# speedup vs baseline: 1.1067x; 1.1067x over previous
import jax
import jax.numpy as jnp
from jax import lax
from jax.experimental import pallas as pl
from jax.experimental.pallas import tpu as pltpu
from jax.experimental.pallas import tpu_sc as plsc

_D = 64
_SEQ = 200
_BATCH = 4096
_NW = 32
_BB = _BATCH // _NW
_NBUF = 4
_NITER = _SEQ // _NBUF


def _pos_encoding(max_len, d_model):
    angle = jnp.arange(d_model, dtype=jnp.float32)
    angle = 10000.0 ** (2.0 * (angle / d_model))
    angle = jnp.arange(max_len, dtype=jnp.float32)[:, None] / angle
    values = jnp.stack([jnp.sin(angle[:, 0::2]), jnp.cos(angle[:, 1::2])], axis=2)
    return jnp.reshape(values, (values.shape[0], -1)).astype(jnp.float32)


def _sc_embed(seqT_hbm, pos_hbm, table_hbm, out_hbm, pos_v, idx_v,
              rows_g, sw, rows_w, isems, gsems, wsems):
    wid = lax.axis_index("s") * 2 + lax.axis_index("c")
    base_b = wid * _BB
    pltpu.sync_copy(pos_hbm, pos_v)

    iota16 = lax.iota(jnp.int32, 16)

    def fire_idx(s, b):
        pltpu.async_copy(seqT_hbm.at[s, pl.ds(base_b, _BB)], idx_v[b], isems[b])

    def prep(s, b):
        pltpu.make_async_copy(seqT_hbm.at[s, pl.ds(base_b, _BB)], idx_v[b],
                              isems[b]).wait()

        def enq(t, _):
            vec = idx_v[b][pl.ds(t * 16, 16)]
            for j in range(16):
                k = t * 16 + j
                pltpu.async_copy(table_hbm.at[vec[j]], rows_g[b].at[k],
                                 gsems[b])
            return 0

        lax.fori_loop(0, _BB // 16, enq, 0)

    def drain_rows(b):
        pltpu.make_async_copy(table_hbm.at[pl.ds(0, _BB), :], rows_g[b],
                              gsems[b]).wait()

    def madd(s, b):
        pr = [pos_v[pl.ds(s * _D + i * 16, 16)] for i in range(4)]

        @plsc.parallel_loop(0, _BB)
        def _rows(r):
            r64 = r * 64
            for i in range(4):
                dvec = iota16 + (i * 16)
                val = rows_g[b][r, pl.ds(i * 16, 16)] * 8.0 + pr[i]
                addr = r64 + ((dvec + r) & 63)
                plsc.store_scatter(sw, [addr], val)

        @plsc.parallel_loop(0, _D)
        def _dims(d):
            for g in range(8):
                bvec = iota16 + (g * 16)
                addr = bvec * 64 + ((bvec + d) & 63)
                val = plsc.load_gather(sw, [addr])
                rows_w[b][d // 8, d % 8, pl.ds(g * 16, 16)] = val

    def fire_write(s, b):
        pltpu.async_copy(rows_w[b], out_hbm.at[pl.ds(s * 8, 8), wid], wsems[b])

    def wait_write(s, b):
        pltpu.make_async_copy(rows_w[b], out_hbm.at[pl.ds(s * 8, 8), wid],
                              wsems[b]).wait()

    for b in range(_NBUF):
        fire_idx(b, b)
    prep(0, 0)
    prep(1, 1)

    def ring_iter(g, _):
        for b in range(_NBUF):
            s = g * _NBUF + b

            @pl.when(s >= _NBUF)
            def _():
                wait_write(s - _NBUF, b)

            drain_rows(b)
            madd(s, b)
            fire_write(s, b)

            @pl.when(s + 2 < _SEQ)
            def _():
                prep(s + 2, (b + 2) % _NBUF)

            @pl.when(s + _NBUF < _SEQ)
            def _():
                fire_idx(s + _NBUF, b)
        return 0

    lax.fori_loop(0, _NITER, ring_iter, 0)
    for b in range(_NBUF):
        wait_write(_SEQ - _NBUF + b, b)


@jax.jit
def _embed(sequences, table):
    pos = _pos_encoding(_SEQ, _D).reshape(_SEQ * _D)
    seq_t = sequences.astype(jnp.int32).T
    mesh = plsc.VectorSubcoreMesh(core_axis_name="c", subcore_axis_name="s")
    out = pl.kernel(
        _sc_embed,
        out_type=jax.ShapeDtypeStruct((_SEQ * 8, _NW, 8, 128), jnp.float32),
        mesh=mesh,
        scratch_types=[
            pltpu.VMEM((_SEQ * _D,), jnp.float32),
            [pltpu.VMEM((_BB,), jnp.int32) for _ in range(_NBUF)],
            [pltpu.VMEM((_BB, _D), jnp.float32) for _ in range(_NBUF)],
            pltpu.VMEM((_BB * 64,), jnp.float32),
            [pltpu.VMEM((8, 8, 128), jnp.float32) for _ in range(_NBUF)],
            [pltpu.SemaphoreType.DMA for _ in range(_NBUF)],
            [pltpu.SemaphoreType.DMA for _ in range(_NBUF)],
            [pltpu.SemaphoreType.DMA for _ in range(_NBUF)],
        ],
        compiler_params=pltpu.CompilerParams(use_tc_tiling_on_sc=True,
                                             needs_layout_passes=False),
    )(seq_t, pos, table)
    out5 = out.reshape(_SEQ, 8, _NW, 8, 128)
    return out5.transpose(2, 4, 0, 1, 3).reshape(_BATCH, _SEQ, _D)


def kernel(sequences, table):
    return _embed(sequences, table)


# speedup vs baseline: 1.1073x; 1.0005x over previous
import jax
import jax.numpy as jnp
from jax import lax
from jax.experimental import pallas as pl
from jax.experimental.pallas import tpu as pltpu
from jax.experimental.pallas import tpu_sc as plsc

_D = 64
_SEQ = 200
_BATCH = 4096
_NW = 32
_BB = _BATCH // _NW
_NBUF = 4
_NITER = _SEQ // _NBUF


def _pos_encoding(max_len, d_model):
    angle = jnp.arange(d_model, dtype=jnp.float32)
    angle = 10000.0 ** (2.0 * (angle / d_model))
    angle = jnp.arange(max_len, dtype=jnp.float32)[:, None] / angle
    values = jnp.stack([jnp.sin(angle[:, 0::2]), jnp.cos(angle[:, 1::2])], axis=2)
    return jnp.reshape(values, (values.shape[0], -1)).astype(jnp.float32)


def _sc_embed(seqT_hbm, pos_hbm, table_hbm, out_hbm, pos_v, idx_v,
              rows_g, sw, rows_w, isems, gsems, wsems):
    wid = lax.axis_index("s") * 2 + lax.axis_index("c")
    base_b = wid * _BB
    pltpu.sync_copy(pos_hbm, pos_v)

    iota16 = lax.iota(jnp.int32, 16)

    def fire_idx(s, b):
        pltpu.async_copy(seqT_hbm.at[s, pl.ds(base_b, _BB)], idx_v[b], isems[b])

    def prep(s, b):
        pltpu.make_async_copy(seqT_hbm.at[s, pl.ds(base_b, _BB)], idx_v[b],
                              isems[b]).wait()

        @plsc.parallel_loop(0, _BB // 16)
        def _enq(t):
            vec = idx_v[b][pl.ds(t * 16, 16)]
            for j in range(16):
                k = t * 16 + j
                pltpu.async_copy(table_hbm.at[vec[j]], rows_g[b].at[k],
                                 gsems[b])

    def drain_rows(b):
        pltpu.make_async_copy(table_hbm.at[pl.ds(0, _BB), :], rows_g[b],
                              gsems[b]).wait()

    def madd(s, b):
        pr = [pos_v[pl.ds(s * _D + i * 16, 16)] for i in range(4)]

        @plsc.parallel_loop(0, _BB)
        def _rows(r):
            r64 = r * 64
            for i in range(4):
                dvec = iota16 + (i * 16)
                val = rows_g[b][r, pl.ds(i * 16, 16)] * 8.0 + pr[i]
                addr = r64 + ((dvec + r) & 63)
                plsc.store_scatter(sw, [addr], val)

        @plsc.parallel_loop(0, _D)
        def _dims(d):
            for g in range(8):
                bvec = iota16 + (g * 16)
                addr = bvec * 64 + ((bvec + d) & 63)
                val = plsc.load_gather(sw, [addr])
                rows_w[b][d // 8, d % 8, pl.ds(g * 16, 16)] = val

    def fire_write(s, b):
        pltpu.async_copy(rows_w[b], out_hbm.at[pl.ds(s * 8, 8), wid], wsems[b])

    def wait_write(s, b):
        pltpu.make_async_copy(rows_w[b], out_hbm.at[pl.ds(s * 8, 8), wid],
                              wsems[b]).wait()

    for b in range(_NBUF):
        fire_idx(b, b)
    prep(0, 0)
    prep(1, 1)

    def ring_iter(g, _):
        for b in range(_NBUF):
            s = g * _NBUF + b

            @pl.when(s >= _NBUF)
            def _():
                wait_write(s - _NBUF, b)

            drain_rows(b)
            madd(s, b)
            fire_write(s, b)

            @pl.when(s + 2 < _SEQ)
            def _():
                prep(s + 2, (b + 2) % _NBUF)

            @pl.when(s + _NBUF < _SEQ)
            def _():
                fire_idx(s + _NBUF, b)
        return 0

    lax.fori_loop(0, _NITER, ring_iter, 0)
    for b in range(_NBUF):
        wait_write(_SEQ - _NBUF + b, b)


@jax.jit
def _embed(sequences, table):
    pos = _pos_encoding(_SEQ, _D).reshape(_SEQ * _D)
    seq_t = sequences.astype(jnp.int32).T
    mesh = plsc.VectorSubcoreMesh(core_axis_name="c", subcore_axis_name="s")
    out = pl.kernel(
        _sc_embed,
        out_type=jax.ShapeDtypeStruct((_SEQ * 8, _NW, 8, 128), jnp.float32),
        mesh=mesh,
        scratch_types=[
            pltpu.VMEM((_SEQ * _D,), jnp.float32),
            [pltpu.VMEM((_BB,), jnp.int32) for _ in range(_NBUF)],
            [pltpu.VMEM((_BB, _D), jnp.float32) for _ in range(_NBUF)],
            pltpu.VMEM((_BB * 64,), jnp.float32),
            [pltpu.VMEM((8, 8, 128), jnp.float32) for _ in range(_NBUF)],
            [pltpu.SemaphoreType.DMA for _ in range(_NBUF)],
            [pltpu.SemaphoreType.DMA for _ in range(_NBUF)],
            [pltpu.SemaphoreType.DMA for _ in range(_NBUF)],
        ],
        compiler_params=pltpu.CompilerParams(use_tc_tiling_on_sc=True,
                                             needs_layout_passes=False),
    )(seq_t, pos, table)
    out5 = out.reshape(_SEQ, 8, _NW, 8, 128)
    return out5.transpose(2, 4, 0, 1, 3).reshape(_BATCH, _SEQ, _D)


def kernel(sequences, table):
    return _embed(sequences, table)


# speedup vs baseline: 1.1086x; 1.0012x over previous
import jax
import jax.numpy as jnp
from jax import lax
from jax.experimental import pallas as pl
from jax.experimental.pallas import tpu as pltpu
from jax.experimental.pallas import tpu_sc as plsc

_D = 64
_SEQ = 200
_BATCH = 4096
_NW = 32
_BB = _BATCH // _NW
_NBUF = 4
_NITER = _SEQ // _NBUF


def _pos_encoding(max_len, d_model):
    angle = jnp.arange(d_model, dtype=jnp.float32)
    angle = 10000.0 ** (2.0 * (angle / d_model))
    angle = jnp.arange(max_len, dtype=jnp.float32)[:, None] / angle
    values = jnp.stack([jnp.sin(angle[:, 0::2]), jnp.cos(angle[:, 1::2])], axis=2)
    return jnp.reshape(values, (values.shape[0], -1)).astype(jnp.float32)


def _sc_embed(seqT_hbm, pos_hbm, table_hbm, out_hbm, pos_v, idx_v,
              rows_g, sw, rows_w, isems, gsems, wsems):
    wid = lax.axis_index("s") * 2 + lax.axis_index("c")
    base_b = wid * _BB
    pltpu.sync_copy(pos_hbm, pos_v)

    iota16 = lax.iota(jnp.int32, 16)

    def fire_idx(s, b):
        pltpu.async_copy(seqT_hbm.at[s, pl.ds(base_b, _BB)], idx_v[b], isems[b])

    def prep(s, b):
        pltpu.make_async_copy(seqT_hbm.at[s, pl.ds(base_b, _BB)], idx_v[b],
                              isems[b]).wait()

        @plsc.parallel_loop(0, _BB // 16)
        def _enq(t):
            vec = idx_v[b][pl.ds(t * 16, 16)]
            for j in range(16):
                k = t * 16 + j
                pltpu.async_copy(table_hbm.at[vec[j]], rows_g[b].at[k],
                                 gsems[b])

    def drain_rows(b):
        pltpu.make_async_copy(table_hbm.at[pl.ds(0, _BB), :], rows_g[b],
                              gsems[b]).wait()

    def madd(s, b):
        pr = [pos_v[pl.ds(s * _D + i * 16, 16)] for i in range(4)]

        @plsc.parallel_loop(0, _BB, unroll=2)
        def _rows(r):
            r64 = r * 64
            for i in range(4):
                dvec = iota16 + (i * 16)
                val = rows_g[b][r, pl.ds(i * 16, 16)] * 8.0 + pr[i]
                addr = r64 + ((dvec + r) & 63)
                plsc.store_scatter(sw, [addr], val)

        @plsc.parallel_loop(0, _D, unroll=2)
        def _dims(d):
            for g in range(8):
                bvec = iota16 + (g * 16)
                addr = bvec * 64 + ((bvec + d) & 63)
                val = plsc.load_gather(sw, [addr])
                rows_w[b][d // 8, d % 8, pl.ds(g * 16, 16)] = val

    def fire_write(s, b):
        pltpu.async_copy(rows_w[b], out_hbm.at[pl.ds(s * 8, 8), wid], wsems[b])

    def wait_write(s, b):
        pltpu.make_async_copy(rows_w[b], out_hbm.at[pl.ds(s * 8, 8), wid],
                              wsems[b]).wait()

    for b in range(_NBUF):
        fire_idx(b, b)
    prep(0, 0)
    prep(1, 1)

    def ring_iter(g, _):
        for b in range(_NBUF):
            s = g * _NBUF + b

            @pl.when(s >= _NBUF)
            def _():
                wait_write(s - _NBUF, b)

            drain_rows(b)
            madd(s, b)
            fire_write(s, b)

            @pl.when(s + 2 < _SEQ)
            def _():
                prep(s + 2, (b + 2) % _NBUF)

            @pl.when(s + _NBUF < _SEQ)
            def _():
                fire_idx(s + _NBUF, b)
        return 0

    lax.fori_loop(0, _NITER, ring_iter, 0)
    for b in range(_NBUF):
        wait_write(_SEQ - _NBUF + b, b)


@jax.jit
def _embed(sequences, table):
    pos = _pos_encoding(_SEQ, _D).reshape(_SEQ * _D)
    seq_t = sequences.astype(jnp.int32).T
    mesh = plsc.VectorSubcoreMesh(core_axis_name="c", subcore_axis_name="s")
    out = pl.kernel(
        _sc_embed,
        out_type=jax.ShapeDtypeStruct((_SEQ * 8, _NW, 8, 128), jnp.float32),
        mesh=mesh,
        scratch_types=[
            pltpu.VMEM((_SEQ * _D,), jnp.float32),
            [pltpu.VMEM((_BB,), jnp.int32) for _ in range(_NBUF)],
            [pltpu.VMEM((_BB, _D), jnp.float32) for _ in range(_NBUF)],
            pltpu.VMEM((_BB * 64,), jnp.float32),
            [pltpu.VMEM((8, 8, 128), jnp.float32) for _ in range(_NBUF)],
            [pltpu.SemaphoreType.DMA for _ in range(_NBUF)],
            [pltpu.SemaphoreType.DMA for _ in range(_NBUF)],
            [pltpu.SemaphoreType.DMA for _ in range(_NBUF)],
        ],
        compiler_params=pltpu.CompilerParams(use_tc_tiling_on_sc=True,
                                             needs_layout_passes=False),
    )(seq_t, pos, table)
    out5 = out.reshape(_SEQ, 8, _NW, 8, 128)
    return out5.transpose(2, 4, 0, 1, 3).reshape(_BATCH, _SEQ, _D)


def kernel(sequences, table):
    return _embed(sequences, table)
